# baseline (speedup 1.0000x reference)
import functools

import jax
import jax.numpy as jnp
from jax import lax
from jax.experimental import pallas as pl
from jax.experimental.pallas import tpu as pltpu
from jax.experimental.pallas import tpu_sc as plsc

N = 4096
K = 33
KO = 48
R2 = 25.0
ROWS = 512
RPW = 128
CAND = N + 16
INF = float("inf")
BIGI = 2**30


def _dist_body(pos8_ref, posT_ref, sq_ref, sqT_ref, b_ref, bT_ref, d_ref):
    blk = pl.program_id(0)
    dot = jnp.dot(pos8_ref[...], posT_ref[...],
                  preferred_element_type=jnp.float32)
    d2 = sq_ref[...] + sqT_ref[...] - 2.0 * dot
    d2 = jnp.maximum(d2, 0.0)
    row_ids = blk * ROWS + jax.lax.broadcasted_iota(jnp.int32, (ROWS, N), 0)
    col_ids = jax.lax.broadcasted_iota(jnp.int32, (ROWS, N), 1)
    valid = (b_ref[...] == bT_ref[...]) & (row_ids != col_ids) & (d2 <= R2)
    d_ref[...] = jnp.where(valid, d2, INF)


def _sqrt_body(x_ref, o_ref):
    o_ref[...] = jnp.sqrt(x_ref[...])


def _sc_topk_body(d_hbm, px_hbm, py_hbm, pz_hbm,
                  src_hbm, tgt_hbm, vx_hbm, vy_hbm, vz_hbm, sqe_hbm,
                  rb0, rb1, cand_i, px_v, py_v, pz_v,
                  osrc, otgt, ovx, ovy, ovz, osqe,
                  sem0, sem1):
    nc = 2
    wid = lax.axis_index("s") * nc + lax.axis_index("c")
    base = wid * RPW
    iota16 = lax.iota(jnp.int32, 16)

    pltpu.sync_copy(px_hbm, px_v)
    pltpu.sync_copy(py_hbm, py_v)
    pltpu.sync_copy(pz_hbm, pz_v)

    inf16 = jnp.full((16,), INF, jnp.float32)
    rb0[pl.ds(N, 16)] = inf16
    rb1[pl.ds(N, 16)] = inf16
    neg16 = jnp.full((16,), -1, jnp.int32)
    lane0 = iota16 == 0

    def process(rb, r, ri):
        @plsc.parallel_loop(0, N // 16, carry=jnp.int32(0), unroll=16)
        def cnt(c, cnt):
            v = rb[pl.ds(c * 16, 16)]
            m = v <= R2
            mi = m.astype(jnp.int32)
            pos = cnt + plsc.cumsum(mi) - 1
            plsc.store_scatter(cand_i, [pos], c * 16 + iota16, mask=m)
            return cnt + jnp.sum(mi)
        padpos = cnt + iota16
        plsc.store_scatter(cand_i, [padpos], N + iota16)
        nv = (cnt + 15) // 16

        risplat = jnp.full((16,), ri, jnp.int32)
        osrc[ri, pl.ds(0, 16)] = neg16
        osrc[ri, pl.ds(16, 16)] = neg16
        osrc[ri, pl.ds(32, 16)] = neg16

        def ext_body(k, _):
            @plsc.parallel_loop(0, nv, carry=(jnp.float32(INF),
                                              jnp.int32(BIGI)), unroll=4)
            def min_carry(t, carry):
                m, bi = carry
                vi = cand_i[pl.ds(t * 16, 16)]
                vd = plsc.load_gather(rb, [vi])
                ml = jnp.min(vd)
                bl = jnp.min(jnp.where(vd == ml, vi, BIGI))
                lt = ml < m
                eq = ml == m
                return (jnp.where(lt, ml, m),
                        jnp.where(lt, bl,
                                  jnp.where(eq, jnp.minimum(bi, bl), bi)))
            m, bi = min_carry
            bsplat = jnp.full((16,), bi, jnp.int32)
            plsc.store_scatter(rb, [bsplat], inf16, mask=lane0)
            plsc.store_scatter(osrc, [risplat, jnp.full((16,), k, jnp.int32)],
                               bsplat, mask=lane0)
            return 0
        lax.fori_loop(0, jnp.minimum(cnt, K), ext_body, 0)

        rsplat = jnp.full((16,), r, jnp.int32)
        rx = plsc.load_gather(px_v, [rsplat])
        ry = plsc.load_gather(py_v, [rsplat])
        rz = plsc.load_gather(pz_v, [rsplat])
        for t in range(KO // 16):
            sidx = osrc[ri, pl.ds(t * 16, 16)]
            val = sidx >= 0
            ci = jnp.maximum(sidx, 0)
            gx = plsc.load_gather(px_v, [ci])
            gy = plsc.load_gather(py_v, [ci])
            gz = plsc.load_gather(pz_v, [ci])
            zeros = jnp.zeros((16,), jnp.float32)
            vx = jnp.where(val, gx - rx, zeros)
            vy = jnp.where(val, gy - ry, zeros)
            vz = jnp.where(val, gz - rz, zeros)
            sq = vx * vx + vy * vy + vz * vz
            ovx[ri, pl.ds(t * 16, 16)] = vx
            ovy[ri, pl.ds(t * 16, 16)] = vy
            ovz[ri, pl.ds(t * 16, 16)] = vz
            osqe[ri, pl.ds(t * 16, 16)] = sq
            otgt[ri, pl.ds(t * 16, 16)] = jnp.where(val, r, -1)

    pltpu.async_copy(d_hbm.at[base], rb0.at[pl.ds(0, N)], sem0)
    def row_step(i, _):
        r0 = base + 2 * i
        pltpu.make_async_copy(d_hbm.at[r0], rb0.at[pl.ds(0, N)], sem0).wait()
        r1 = r0 + 1
        pltpu.async_copy(d_hbm.at[r1], rb1.at[pl.ds(0, N)], sem1)
        process(rb0, r0, 2 * i)
        pltpu.make_async_copy(d_hbm.at[r1], rb1.at[pl.ds(0, N)], sem1).wait()
        r2 = jnp.minimum(r0 + 2, N - 1)
        pltpu.async_copy(d_hbm.at[r2], rb0.at[pl.ds(0, N)], sem0)
        process(rb1, r1, 2 * i + 1)
        return 0
    lax.fori_loop(0, RPW // 2, row_step, 0)
    pltpu.make_async_copy(d_hbm.at[0], rb0.at[pl.ds(0, N)], sem0).wait()

    pltpu.sync_copy(osrc, src_hbm.at[pl.ds(base, RPW)])
    pltpu.sync_copy(otgt, tgt_hbm.at[pl.ds(base, RPW)])
    pltpu.sync_copy(ovx, vx_hbm.at[pl.ds(base, RPW)])
    pltpu.sync_copy(ovy, vy_hbm.at[pl.ds(base, RPW)])
    pltpu.sync_copy(ovz, vz_hbm.at[pl.ds(base, RPW)])
    pltpu.sync_copy(osqe, sqe_hbm.at[pl.ds(base, RPW)])


def _sc_topk(d, px, py, pz):
    mesh = plsc.VectorSubcoreMesh(core_axis_name="c", subcore_axis_name="s")
    f32 = jnp.float32
    i32 = jnp.int32
    fn = functools.partial(
        pl.kernel,
        mesh=mesh,
        compiler_params=pltpu.CompilerParams(needs_layout_passes=False),
        out_type=[
            jax.ShapeDtypeStruct((N, KO), i32),
            jax.ShapeDtypeStruct((N, KO), i32),
            jax.ShapeDtypeStruct((N, KO), f32),
            jax.ShapeDtypeStruct((N, KO), f32),
            jax.ShapeDtypeStruct((N, KO), f32),
            jax.ShapeDtypeStruct((N, KO), f32),
        ],
        scratch_types=[
            pltpu.VMEM((CAND,), f32),
            pltpu.VMEM((CAND,), f32),
            pltpu.VMEM((CAND,), i32),
            pltpu.VMEM((N,), f32),
            pltpu.VMEM((N,), f32),
            pltpu.VMEM((N,), f32),
            pltpu.VMEM((RPW, KO), i32),
            pltpu.VMEM((RPW, KO), i32),
            pltpu.VMEM((RPW, KO), f32),
            pltpu.VMEM((RPW, KO), f32),
            pltpu.VMEM((RPW, KO), f32),
            pltpu.VMEM((RPW, KO), f32),
            pltpu.SemaphoreType.DMA,
            pltpu.SemaphoreType.DMA,
        ],
    )(_sc_topk_body)
    return fn(d, px, py, pz)


@jax.jit
def kernel(pos, batch):
    n = pos.shape[0]
    sq = jnp.sum(pos * pos, axis=1)
    pos8 = jnp.pad(pos, ((0, 0), (0, 5)))
    posT = pos8.T
    d = pl.pallas_call(
        _dist_body,
        grid=(n // ROWS,),
        in_specs=[
            pl.BlockSpec((ROWS, 8), lambda i: (i, 0)),
            pl.BlockSpec((8, N), lambda i: (0, 0)),
            pl.BlockSpec((ROWS, 1), lambda i: (i, 0)),
            pl.BlockSpec((1, N), lambda i: (0, 0)),
            pl.BlockSpec((ROWS, 1), lambda i: (i, 0)),
            pl.BlockSpec((1, N), lambda i: (0, 0)),
        ],
        out_specs=pl.BlockSpec((ROWS, N), lambda i: (i, 0)),
        out_shape=jax.ShapeDtypeStruct((n, n), jnp.float32),
    )(pos8, posT, sq[:, None], sq[None, :], batch[:, None], batch[None, :])

    px = jnp.asarray(pos[:, 0])
    py = jnp.asarray(pos[:, 1])
    pz = jnp.asarray(pos[:, 2])
    src, tgt, vx, vy, vz, sqe = _sc_topk(d, px, py, pz)

    w = pl.pallas_call(
        _sqrt_body,
        out_shape=jax.ShapeDtypeStruct((n, KO), jnp.float32),
    )(sqe)

    src = src[:, :K].reshape(-1)
    tgt = tgt[:, :K].reshape(-1)
    edge_index = jnp.stack([src, tgt])
    edge_weight = w[:, :K].reshape(-1)
    edge_vec = jnp.stack([vx[:, :K], vy[:, :K], vz[:, :K]],
                         axis=-1).reshape(-1, 3)
    return edge_index, edge_weight, edge_vec


# speedup vs baseline: 1.4070x; 1.4070x over previous
import functools

import jax
import jax.numpy as jnp
from jax import lax
from jax.experimental import pallas as pl
from jax.experimental.pallas import tpu as pltpu
from jax.experimental.pallas import tpu_sc as plsc

N = 4096
K = 33
KO = 48
R2 = 25.0
ROWS = 512
RPW = 128
CAND = N + 16
INF = float("inf")
BIGI = 2**30


def _dist_body(pos8_ref, posT_ref, sq_ref, sqT_ref, b_ref, bT_ref, d_ref):
    blk = pl.program_id(0)
    dot = jnp.dot(pos8_ref[...], posT_ref[...],
                  preferred_element_type=jnp.float32)
    d2 = sq_ref[...] + sqT_ref[...] - 2.0 * dot
    d2 = jnp.maximum(d2, 0.0)
    row_ids = blk * ROWS + jax.lax.broadcasted_iota(jnp.int32, (ROWS, N), 0)
    col_ids = jax.lax.broadcasted_iota(jnp.int32, (ROWS, N), 1)
    valid = (b_ref[...] == bT_ref[...]) & (row_ids != col_ids) & (d2 <= R2)
    d_ref[...] = jnp.where(valid, d2, INF)


def _sqrt_body(x_ref, o_ref):
    o_ref[...] = jnp.sqrt(x_ref[...])


def _sc_topk_body(d_hbm, px_hbm, py_hbm, pz_hbm,
                  src_hbm, tgt_hbm, vx_hbm, vy_hbm, vz_hbm, sqe_hbm,
                  rb0, rb1, cand_i, px_v, py_v, pz_v,
                  osrc, otgt, ovx, ovy, ovz, osqe,
                  sem0, sem1):
    nc = 2
    wid = lax.axis_index("s") * nc + lax.axis_index("c")
    base = wid * RPW
    iota16 = lax.iota(jnp.int32, 16)

    pltpu.sync_copy(px_hbm, px_v)
    pltpu.sync_copy(py_hbm, py_v)
    pltpu.sync_copy(pz_hbm, pz_v)

    inf16 = jnp.full((16,), INF, jnp.float32)
    rb0[pl.ds(N, 16)] = inf16
    rb1[pl.ds(N, 16)] = inf16
    neg16 = jnp.full((16,), -1, jnp.int32)
    lane0 = iota16 == 0

    def process(rb, r, ri):
        @plsc.parallel_loop(0, N // 16, carry=jnp.int32(0), unroll=8)
        def cnt(c, cnt):
            v = rb[pl.ds(c * 16, 16)]
            m = v <= R2
            cs = plsc.cumsum(m.astype(jnp.int32))
            pos = cnt + cs - 1
            plsc.store_scatter(cand_i, [pos], c * 16 + iota16, mask=m)
            return cnt + cs[15]
        padpos = cnt + iota16
        plsc.store_scatter(cand_i, [padpos], N + iota16)
        nv = (cnt + 15) // 16

        risplat = jnp.full((16,), ri, jnp.int32)
        osrc[ri, pl.ds(0, 16)] = neg16
        osrc[ri, pl.ds(16, 16)] = neg16
        osrc[ri, pl.ds(32, 16)] = neg16

        def ext_body(k, _):
            def min_body(t, carry):
                m, bi = carry
                vi = cand_i[pl.ds(t * 16, 16)]
                vd = plsc.load_gather(rb, [vi])
                ml = jnp.min(vd)
                ffs = plsc.all_reduce_ffs(vd == ml)
                bl = plsc.load_gather(cand_i,
                                      [t * 16 + ffs])[0]
                lt = ml < m
                eq = ml == m
                return (jnp.where(lt, ml, m),
                        jnp.where(lt, bl,
                                  jnp.where(eq, jnp.minimum(bi, bl), bi)))
            m, bi = lax.fori_loop(0, nv, min_body,
                                  (jnp.float32(INF), jnp.int32(BIGI)))
            bsplat = jnp.full((16,), bi, jnp.int32)
            plsc.store_scatter(rb, [bsplat], inf16, mask=lane0)
            plsc.store_scatter(osrc, [risplat, jnp.full((16,), k, jnp.int32)],
                               bsplat, mask=lane0)
            return 0
        lax.fori_loop(0, jnp.minimum(cnt, K), ext_body, 0)

        rsplat = jnp.full((16,), r, jnp.int32)
        rx = plsc.load_gather(px_v, [rsplat])
        ry = plsc.load_gather(py_v, [rsplat])
        rz = plsc.load_gather(pz_v, [rsplat])
        for t in range(KO // 16):
            sidx = osrc[ri, pl.ds(t * 16, 16)]
            val = sidx >= 0
            ci = jnp.maximum(sidx, 0)
            gx = plsc.load_gather(px_v, [ci])
            gy = plsc.load_gather(py_v, [ci])
            gz = plsc.load_gather(pz_v, [ci])
            zeros = jnp.zeros((16,), jnp.float32)
            vx = jnp.where(val, gx - rx, zeros)
            vy = jnp.where(val, gy - ry, zeros)
            vz = jnp.where(val, gz - rz, zeros)
            sq = vx * vx + vy * vy + vz * vz
            ovx[ri, pl.ds(t * 16, 16)] = vx
            ovy[ri, pl.ds(t * 16, 16)] = vy
            ovz[ri, pl.ds(t * 16, 16)] = vz
            osqe[ri, pl.ds(t * 16, 16)] = sq
            otgt[ri, pl.ds(t * 16, 16)] = jnp.where(val, r, -1)

    pltpu.async_copy(d_hbm.at[base], rb0.at[pl.ds(0, N)], sem0)
    def row_step(i, _):
        r0 = base + 2 * i
        pltpu.make_async_copy(d_hbm.at[r0], rb0.at[pl.ds(0, N)], sem0).wait()
        r1 = r0 + 1
        pltpu.async_copy(d_hbm.at[r1], rb1.at[pl.ds(0, N)], sem1)
        process(rb0, r0, 2 * i)
        pltpu.make_async_copy(d_hbm.at[r1], rb1.at[pl.ds(0, N)], sem1).wait()
        r2 = jnp.minimum(r0 + 2, N - 1)
        pltpu.async_copy(d_hbm.at[r2], rb0.at[pl.ds(0, N)], sem0)
        process(rb1, r1, 2 * i + 1)
        return 0
    lax.fori_loop(0, RPW // 2, row_step, 0)
    pltpu.make_async_copy(d_hbm.at[0], rb0.at[pl.ds(0, N)], sem0).wait()

    pltpu.sync_copy(osrc, src_hbm.at[pl.ds(base, RPW)])
    pltpu.sync_copy(otgt, tgt_hbm.at[pl.ds(base, RPW)])
    pltpu.sync_copy(ovx, vx_hbm.at[pl.ds(base, RPW)])
    pltpu.sync_copy(ovy, vy_hbm.at[pl.ds(base, RPW)])
    pltpu.sync_copy(ovz, vz_hbm.at[pl.ds(base, RPW)])
    pltpu.sync_copy(osqe, sqe_hbm.at[pl.ds(base, RPW)])


def _sc_topk(d, px, py, pz):
    mesh = plsc.VectorSubcoreMesh(core_axis_name="c", subcore_axis_name="s")
    f32 = jnp.float32
    i32 = jnp.int32
    fn = functools.partial(
        pl.kernel,
        mesh=mesh,
        compiler_params=pltpu.CompilerParams(needs_layout_passes=False),
        out_type=[
            jax.ShapeDtypeStruct((N, KO), i32),
            jax.ShapeDtypeStruct((N, KO), i32),
            jax.ShapeDtypeStruct((N, KO), f32),
            jax.ShapeDtypeStruct((N, KO), f32),
            jax.ShapeDtypeStruct((N, KO), f32),
            jax.ShapeDtypeStruct((N, KO), f32),
        ],
        scratch_types=[
            pltpu.VMEM((CAND,), f32),
            pltpu.VMEM((CAND,), f32),
            pltpu.VMEM((CAND,), i32),
            pltpu.VMEM((N,), f32),
            pltpu.VMEM((N,), f32),
            pltpu.VMEM((N,), f32),
            pltpu.VMEM((RPW, KO), i32),
            pltpu.VMEM((RPW, KO), i32),
            pltpu.VMEM((RPW, KO), f32),
            pltpu.VMEM((RPW, KO), f32),
            pltpu.VMEM((RPW, KO), f32),
            pltpu.VMEM((RPW, KO), f32),
            pltpu.SemaphoreType.DMA,
            pltpu.SemaphoreType.DMA,
        ],
    )(_sc_topk_body)
    return fn(d, px, py, pz)


@jax.jit
def kernel(pos, batch):
    n = pos.shape[0]
    sq = jnp.sum(pos * pos, axis=1)
    pos8 = jnp.pad(pos, ((0, 0), (0, 5)))
    posT = pos8.T
    d = pl.pallas_call(
        _dist_body,
        grid=(n // ROWS,),
        in_specs=[
            pl.BlockSpec((ROWS, 8), lambda i: (i, 0)),
            pl.BlockSpec((8, N), lambda i: (0, 0)),
            pl.BlockSpec((ROWS, 1), lambda i: (i, 0)),
            pl.BlockSpec((1, N), lambda i: (0, 0)),
            pl.BlockSpec((ROWS, 1), lambda i: (i, 0)),
            pl.BlockSpec((1, N), lambda i: (0, 0)),
        ],
        out_specs=pl.BlockSpec((ROWS, N), lambda i: (i, 0)),
        out_shape=jax.ShapeDtypeStruct((n, n), jnp.float32),
    )(pos8, posT, sq[:, None], sq[None, :], batch[:, None], batch[None, :])

    px = jnp.asarray(pos[:, 0])
    py = jnp.asarray(pos[:, 1])
    pz = jnp.asarray(pos[:, 2])
    src, tgt, vx, vy, vz, sqe = _sc_topk(d, px, py, pz)

    w = pl.pallas_call(
        _sqrt_body,
        out_shape=jax.ShapeDtypeStruct((n, KO), jnp.float32),
    )(sqe)

    src = src[:, :K].reshape(-1)
    tgt = tgt[:, :K].reshape(-1)
    edge_index = jnp.stack([src, tgt])
    edge_weight = w[:, :K].reshape(-1)
    edge_vec = jnp.stack([vx[:, :K], vy[:, :K], vz[:, :K]],
                         axis=-1).reshape(-1, 3)
    return edge_index, edge_weight, edge_vec


# speedup vs baseline: 1.5330x; 1.0896x over previous
import functools

import jax
import jax.numpy as jnp
from jax import lax
from jax.experimental import pallas as pl
from jax.experimental.pallas import tpu as pltpu
from jax.experimental.pallas import tpu_sc as plsc

N = 4096
K = 33
KO = 48
R2 = 25.0
ROWS = 512
RPW = 128
CAND = N + 16
INF = float("inf")
BIGI = 2**30


def _dist_body(pos8_ref, posT_ref, sq_ref, sqT_ref, b_ref, bT_ref, d_ref):
    blk = pl.program_id(0)
    dot = jnp.dot(pos8_ref[...], posT_ref[...],
                  preferred_element_type=jnp.float32)
    d2 = sq_ref[...] + sqT_ref[...] - 2.0 * dot
    d2 = jnp.maximum(d2, 0.0)
    row_ids = blk * ROWS + jax.lax.broadcasted_iota(jnp.int32, (ROWS, N), 0)
    col_ids = jax.lax.broadcasted_iota(jnp.int32, (ROWS, N), 1)
    valid = (b_ref[...] == bT_ref[...]) & (row_ids != col_ids) & (d2 <= R2)
    d_ref[...] = jnp.where(valid, d2, INF)


def _sqrt_body(x_ref, o_ref):
    o_ref[...] = jnp.sqrt(x_ref[...])


def _sc_topk_body(d_hbm, px_hbm, py_hbm, pz_hbm,
                  src_hbm, tgt_hbm, vx_hbm, vy_hbm, vz_hbm, sqe_hbm,
                  rb0, rb1, cand_i, px_v, py_v, pz_v,
                  osrc, otgt, ovx, ovy, ovz, osqe,
                  sem0, sem1):
    nc = 2
    wid = lax.axis_index("s") * nc + lax.axis_index("c")
    base = wid * RPW
    iota16 = lax.iota(jnp.int32, 16)

    pltpu.sync_copy(px_hbm, px_v)
    pltpu.sync_copy(py_hbm, py_v)
    pltpu.sync_copy(pz_hbm, pz_v)

    inf16 = jnp.full((16,), INF, jnp.float32)
    rb0[pl.ds(N, 16)] = inf16
    rb1[pl.ds(N, 16)] = inf16
    neg16 = jnp.full((16,), -1, jnp.int32)
    lane0 = iota16 == 0

    def process(rb, r, ri):
        @plsc.parallel_loop(0, N // 16, carry=jnp.int32(0), unroll=8)
        def cnt(c, cnt):
            v = rb[pl.ds(c * 16, 16)]
            m = v <= R2
            cs = plsc.cumsum(m.astype(jnp.int32))
            pos = cnt + cs - 1
            plsc.store_scatter(cand_i, [pos], c * 16 + iota16, mask=m)
            return cnt + cs[15]
        padpos = cnt + iota16
        plsc.store_scatter(cand_i, [padpos], N + iota16)
        nv = (cnt + 15) // 16

        risplat = jnp.full((16,), ri, jnp.int32)
        osrc[ri, pl.ds(0, 16)] = neg16
        osrc[ri, pl.ds(16, 16)] = neg16
        osrc[ri, pl.ds(32, 16)] = neg16

        def ext_body(k, _):
            def min_body(t, carry):
                m, bi = carry
                vi = cand_i[pl.ds(t * 16, 16)]
                vd = plsc.load_gather(rb, [vi])
                ml = jnp.min(vd)
                bl = jnp.min(jnp.where(vd == ml, vi, BIGI))
                lt = ml < m
                eq = ml == m
                return (jnp.where(lt, ml, m),
                        jnp.where(lt, bl,
                                  jnp.where(eq, jnp.minimum(bi, bl), bi)))
            m, bi = lax.fori_loop(0, nv, min_body,
                                  (jnp.float32(INF), jnp.int32(BIGI)))
            bsplat = jnp.full((16,), bi, jnp.int32)
            plsc.store_scatter(rb, [bsplat], inf16, mask=lane0)
            plsc.store_scatter(osrc, [risplat, jnp.full((16,), k, jnp.int32)],
                               bsplat, mask=lane0)
            return 0
        lax.fori_loop(0, jnp.minimum(cnt, K), ext_body, 0)

        rsplat = jnp.full((16,), r, jnp.int32)
        rx = plsc.load_gather(px_v, [rsplat])
        ry = plsc.load_gather(py_v, [rsplat])
        rz = plsc.load_gather(pz_v, [rsplat])
        for t in range(KO // 16):
            sidx = osrc[ri, pl.ds(t * 16, 16)]
            val = sidx >= 0
            ci = jnp.maximum(sidx, 0)
            gx = plsc.load_gather(px_v, [ci])
            gy = plsc.load_gather(py_v, [ci])
            gz = plsc.load_gather(pz_v, [ci])
            zeros = jnp.zeros((16,), jnp.float32)
            vx = jnp.where(val, gx - rx, zeros)
            vy = jnp.where(val, gy - ry, zeros)
            vz = jnp.where(val, gz - rz, zeros)
            sq = vx * vx + vy * vy + vz * vz
            ovx[ri, pl.ds(t * 16, 16)] = vx
            ovy[ri, pl.ds(t * 16, 16)] = vy
            ovz[ri, pl.ds(t * 16, 16)] = vz
            osqe[ri, pl.ds(t * 16, 16)] = sq
            otgt[ri, pl.ds(t * 16, 16)] = jnp.where(val, r, -1)

    pltpu.async_copy(d_hbm.at[base], rb0.at[pl.ds(0, N)], sem0)
    def row_step(i, _):
        r0 = base + 2 * i
        pltpu.make_async_copy(d_hbm.at[r0], rb0.at[pl.ds(0, N)], sem0).wait()
        r1 = r0 + 1
        pltpu.async_copy(d_hbm.at[r1], rb1.at[pl.ds(0, N)], sem1)
        process(rb0, r0, 2 * i)
        pltpu.make_async_copy(d_hbm.at[r1], rb1.at[pl.ds(0, N)], sem1).wait()
        r2 = jnp.minimum(r0 + 2, N - 1)
        pltpu.async_copy(d_hbm.at[r2], rb0.at[pl.ds(0, N)], sem0)
        process(rb1, r1, 2 * i + 1)
        return 0
    lax.fori_loop(0, RPW // 2, row_step, 0)
    pltpu.make_async_copy(d_hbm.at[0], rb0.at[pl.ds(0, N)], sem0).wait()

    pltpu.sync_copy(osrc, src_hbm.at[pl.ds(base, RPW)])
    pltpu.sync_copy(otgt, tgt_hbm.at[pl.ds(base, RPW)])
    pltpu.sync_copy(ovx, vx_hbm.at[pl.ds(base, RPW)])
    pltpu.sync_copy(ovy, vy_hbm.at[pl.ds(base, RPW)])
    pltpu.sync_copy(ovz, vz_hbm.at[pl.ds(base, RPW)])
    pltpu.sync_copy(osqe, sqe_hbm.at[pl.ds(base, RPW)])


def _sc_topk(d, px, py, pz):
    mesh = plsc.VectorSubcoreMesh(core_axis_name="c", subcore_axis_name="s")
    f32 = jnp.float32
    i32 = jnp.int32
    fn = functools.partial(
        pl.kernel,
        mesh=mesh,
        compiler_params=pltpu.CompilerParams(needs_layout_passes=False),
        out_type=[
            jax.ShapeDtypeStruct((N, KO), i32),
            jax.ShapeDtypeStruct((N, KO), i32),
            jax.ShapeDtypeStruct((N, KO), f32),
            jax.ShapeDtypeStruct((N, KO), f32),
            jax.ShapeDtypeStruct((N, KO), f32),
            jax.ShapeDtypeStruct((N, KO), f32),
        ],
        scratch_types=[
            pltpu.VMEM((CAND,), f32),
            pltpu.VMEM((CAND,), f32),
            pltpu.VMEM((CAND,), i32),
            pltpu.VMEM((N,), f32),
            pltpu.VMEM((N,), f32),
            pltpu.VMEM((N,), f32),
            pltpu.VMEM((RPW, KO), i32),
            pltpu.VMEM((RPW, KO), i32),
            pltpu.VMEM((RPW, KO), f32),
            pltpu.VMEM((RPW, KO), f32),
            pltpu.VMEM((RPW, KO), f32),
            pltpu.VMEM((RPW, KO), f32),
            pltpu.SemaphoreType.DMA,
            pltpu.SemaphoreType.DMA,
        ],
    )(_sc_topk_body)
    return fn(d, px, py, pz)


@jax.jit
def kernel(pos, batch):
    n = pos.shape[0]
    sq = jnp.sum(pos * pos, axis=1)
    pos8 = jnp.pad(pos, ((0, 0), (0, 5)))
    posT = pos8.T
    d = pl.pallas_call(
        _dist_body,
        grid=(n // ROWS,),
        in_specs=[
            pl.BlockSpec((ROWS, 8), lambda i: (i, 0)),
            pl.BlockSpec((8, N), lambda i: (0, 0)),
            pl.BlockSpec((ROWS, 1), lambda i: (i, 0)),
            pl.BlockSpec((1, N), lambda i: (0, 0)),
            pl.BlockSpec((ROWS, 1), lambda i: (i, 0)),
            pl.BlockSpec((1, N), lambda i: (0, 0)),
        ],
        out_specs=pl.BlockSpec((ROWS, N), lambda i: (i, 0)),
        out_shape=jax.ShapeDtypeStruct((n, n), jnp.float32),
    )(pos8, posT, sq[:, None], sq[None, :], batch[:, None], batch[None, :])

    px = jnp.asarray(pos[:, 0])
    py = jnp.asarray(pos[:, 1])
    pz = jnp.asarray(pos[:, 2])
    src, tgt, vx, vy, vz, sqe = _sc_topk(d, px, py, pz)

    w = pl.pallas_call(
        _sqrt_body,
        out_shape=jax.ShapeDtypeStruct((n, KO), jnp.float32),
    )(sqe)

    src = src[:, :K].reshape(-1)
    tgt = tgt[:, :K].reshape(-1)
    edge_index = jnp.stack([src, tgt])
    edge_weight = w[:, :K].reshape(-1)
    edge_vec = jnp.stack([vx[:, :K], vy[:, :K], vz[:, :K]],
                         axis=-1).reshape(-1, 3)
    return edge_index, edge_weight, edge_vec


# speedup vs baseline: 1.7223x; 1.1235x over previous
import functools

import jax
import jax.numpy as jnp
from jax import lax
from jax.experimental import pallas as pl
from jax.experimental.pallas import tpu as pltpu
from jax.experimental.pallas import tpu_sc as plsc

N = 4096
K = 33
KO = 48
R2 = 25.0
ROWS = 512
RPW = 128
CAND = N + 32
INF = float("inf")
BIGI = 2**30


def _dist_body(pos8_ref, posT_ref, sq_ref, sqT_ref, b_ref, bT_ref, d_ref):
    blk = pl.program_id(0)
    dot = jnp.dot(pos8_ref[...], posT_ref[...],
                  preferred_element_type=jnp.float32)
    d2 = sq_ref[...] + sqT_ref[...] - 2.0 * dot
    d2 = jnp.maximum(d2, 0.0)
    row_ids = blk * ROWS + jax.lax.broadcasted_iota(jnp.int32, (ROWS, N), 0)
    col_ids = jax.lax.broadcasted_iota(jnp.int32, (ROWS, N), 1)
    valid = (b_ref[...] == bT_ref[...]) & (row_ids != col_ids) & (d2 <= R2)
    d_ref[...] = jnp.where(valid, d2, INF)


def _sqrt_body(x_ref, o_ref):
    o_ref[...] = jnp.sqrt(x_ref[...])


def _sc_topk_body(d_hbm, px_hbm, py_hbm, pz_hbm,
                  src_hbm, tgt_hbm, vx_hbm, vy_hbm, vz_hbm, sqe_hbm,
                  rb0, rb1, cand_i, px_v, py_v, pz_v,
                  osrc, otgt, ovx, ovy, ovz, osqe,
                  sem0, sem1):
    nc = 2
    wid = lax.axis_index("s") * nc + lax.axis_index("c")
    base = wid * RPW
    iota16 = lax.iota(jnp.int32, 16)

    pltpu.sync_copy(px_hbm, px_v)
    pltpu.sync_copy(py_hbm, py_v)
    pltpu.sync_copy(pz_hbm, pz_v)

    inf16 = jnp.full((16,), INF, jnp.float32)
    rb0[pl.ds(N, 16)] = inf16
    rb0[pl.ds(N + 16, 16)] = inf16
    rb1[pl.ds(N, 16)] = inf16
    rb1[pl.ds(N + 16, 16)] = inf16
    neg16 = jnp.full((16,), -1, jnp.int32)
    lane0 = iota16 == 0

    def process(rb, r, ri):
        @plsc.parallel_loop(0, N // 16, carry=jnp.int32(0), unroll=8)
        def cnt(c, cnt):
            v = rb[pl.ds(c * 16, 16)]
            m = v <= R2
            cs = plsc.cumsum(m.astype(jnp.int32))
            pos = cnt + cs - 1
            plsc.store_scatter(cand_i, [pos], c * 16 + iota16, mask=m)
            return cnt + cs[15]
        padpos = cnt + iota16
        plsc.store_scatter(cand_i, [padpos], N + iota16)
        plsc.store_scatter(cand_i, [padpos + 16], N + 16 + iota16)
        nv = (cnt + 15) // 16

        risplat = jnp.full((16,), ri, jnp.int32)
        osrc[ri, pl.ds(0, 16)] = neg16
        osrc[ri, pl.ds(16, 16)] = neg16
        osrc[ri, pl.ds(32, 16)] = neg16

        nk = jnp.minimum(cnt, K)

        @pl.when(cnt <= 32)
        def _():
            vi0 = cand_i[pl.ds(0, 16)]
            vi1 = cand_i[pl.ds(16, 16)]

            def ext_fast(k, _):
                vd0 = plsc.load_gather(rb, [vi0])
                vd1 = plsc.load_gather(rb, [vi1])
                ml0 = jnp.min(vd0)
                ml1 = jnp.min(vd1)
                bl0 = jnp.min(jnp.where(vd0 == ml0, vi0, BIGI))
                bl1 = jnp.min(jnp.where(vd1 == ml1, vi1, BIGI))
                lt = ml1 < ml0
                eq = ml1 == ml0
                bi = jnp.where(lt, bl1,
                               jnp.where(eq, jnp.minimum(bl0, bl1), bl0))
                bsplat = jnp.full((16,), bi, jnp.int32)
                plsc.store_scatter(rb, [bsplat], inf16, mask=lane0)
                plsc.store_scatter(osrc,
                                   [risplat, jnp.full((16,), k, jnp.int32)],
                                   bsplat, mask=lane0)
                return 0
            lax.fori_loop(0, nk, ext_fast, 0)

        @pl.when(cnt > 32)
        def _():
            def ext_body(k, _):
                def min_body(t, carry):
                    m, bi = carry
                    vi = cand_i[pl.ds(t * 16, 16)]
                    vd = plsc.load_gather(rb, [vi])
                    ml = jnp.min(vd)
                    bl = jnp.min(jnp.where(vd == ml, vi, BIGI))
                    lt = ml < m
                    eq = ml == m
                    return (jnp.where(lt, ml, m),
                            jnp.where(lt, bl,
                                      jnp.where(eq, jnp.minimum(bi, bl), bi)))
                m, bi = lax.fori_loop(0, nv, min_body,
                                      (jnp.float32(INF), jnp.int32(BIGI)))
                bsplat = jnp.full((16,), bi, jnp.int32)
                plsc.store_scatter(rb, [bsplat], inf16, mask=lane0)
                plsc.store_scatter(osrc,
                                   [risplat, jnp.full((16,), k, jnp.int32)],
                                   bsplat, mask=lane0)
                return 0
            lax.fori_loop(0, nk, ext_body, 0)

        rsplat = jnp.full((16,), r, jnp.int32)
        rx = plsc.load_gather(px_v, [rsplat])
        ry = plsc.load_gather(py_v, [rsplat])
        rz = plsc.load_gather(pz_v, [rsplat])
        for t in range(KO // 16):
            sidx = osrc[ri, pl.ds(t * 16, 16)]
            val = sidx >= 0
            ci = jnp.maximum(sidx, 0)
            gx = plsc.load_gather(px_v, [ci])
            gy = plsc.load_gather(py_v, [ci])
            gz = plsc.load_gather(pz_v, [ci])
            zeros = jnp.zeros((16,), jnp.float32)
            vx = jnp.where(val, gx - rx, zeros)
            vy = jnp.where(val, gy - ry, zeros)
            vz = jnp.where(val, gz - rz, zeros)
            sq = vx * vx + vy * vy + vz * vz
            ovx[ri, pl.ds(t * 16, 16)] = vx
            ovy[ri, pl.ds(t * 16, 16)] = vy
            ovz[ri, pl.ds(t * 16, 16)] = vz
            osqe[ri, pl.ds(t * 16, 16)] = sq
            otgt[ri, pl.ds(t * 16, 16)] = jnp.where(val, r, -1)

    pltpu.async_copy(d_hbm.at[base], rb0.at[pl.ds(0, N)], sem0)
    def row_step(i, _):
        r0 = base + 2 * i
        pltpu.make_async_copy(d_hbm.at[r0], rb0.at[pl.ds(0, N)], sem0).wait()
        r1 = r0 + 1
        pltpu.async_copy(d_hbm.at[r1], rb1.at[pl.ds(0, N)], sem1)
        process(rb0, r0, 2 * i)
        pltpu.make_async_copy(d_hbm.at[r1], rb1.at[pl.ds(0, N)], sem1).wait()
        r2 = jnp.minimum(r0 + 2, N - 1)
        pltpu.async_copy(d_hbm.at[r2], rb0.at[pl.ds(0, N)], sem0)
        process(rb1, r1, 2 * i + 1)
        return 0
    lax.fori_loop(0, RPW // 2, row_step, 0)
    pltpu.make_async_copy(d_hbm.at[0], rb0.at[pl.ds(0, N)], sem0).wait()

    pltpu.sync_copy(osrc, src_hbm.at[pl.ds(base, RPW)])
    pltpu.sync_copy(otgt, tgt_hbm.at[pl.ds(base, RPW)])
    pltpu.sync_copy(ovx, vx_hbm.at[pl.ds(base, RPW)])
    pltpu.sync_copy(ovy, vy_hbm.at[pl.ds(base, RPW)])
    pltpu.sync_copy(ovz, vz_hbm.at[pl.ds(base, RPW)])
    pltpu.sync_copy(osqe, sqe_hbm.at[pl.ds(base, RPW)])


def _sc_topk(d, px, py, pz):
    mesh = plsc.VectorSubcoreMesh(core_axis_name="c", subcore_axis_name="s")
    f32 = jnp.float32
    i32 = jnp.int32
    fn = functools.partial(
        pl.kernel,
        mesh=mesh,
        compiler_params=pltpu.CompilerParams(needs_layout_passes=False),
        out_type=[
            jax.ShapeDtypeStruct((N, KO), i32),
            jax.ShapeDtypeStruct((N, KO), i32),
            jax.ShapeDtypeStruct((N, KO), f32),
            jax.ShapeDtypeStruct((N, KO), f32),
            jax.ShapeDtypeStruct((N, KO), f32),
            jax.ShapeDtypeStruct((N, KO), f32),
        ],
        scratch_types=[
            pltpu.VMEM((CAND,), f32),
            pltpu.VMEM((CAND,), f32),
            pltpu.VMEM((CAND,), i32),
            pltpu.VMEM((N,), f32),
            pltpu.VMEM((N,), f32),
            pltpu.VMEM((N,), f32),
            pltpu.VMEM((RPW, KO), i32),
            pltpu.VMEM((RPW, KO), i32),
            pltpu.VMEM((RPW, KO), f32),
            pltpu.VMEM((RPW, KO), f32),
            pltpu.VMEM((RPW, KO), f32),
            pltpu.VMEM((RPW, KO), f32),
            pltpu.SemaphoreType.DMA,
            pltpu.SemaphoreType.DMA,
        ],
    )(_sc_topk_body)
    return fn(d, px, py, pz)


@jax.jit
def kernel(pos, batch):
    n = pos.shape[0]
    sq = jnp.sum(pos * pos, axis=1)
    pos8 = jnp.pad(pos, ((0, 0), (0, 5)))
    posT = pos8.T
    d = pl.pallas_call(
        _dist_body,
        grid=(n // ROWS,),
        in_specs=[
            pl.BlockSpec((ROWS, 8), lambda i: (i, 0)),
            pl.BlockSpec((8, N), lambda i: (0, 0)),
            pl.BlockSpec((ROWS, 1), lambda i: (i, 0)),
            pl.BlockSpec((1, N), lambda i: (0, 0)),
            pl.BlockSpec((ROWS, 1), lambda i: (i, 0)),
            pl.BlockSpec((1, N), lambda i: (0, 0)),
        ],
        out_specs=pl.BlockSpec((ROWS, N), lambda i: (i, 0)),
        out_shape=jax.ShapeDtypeStruct((n, n), jnp.float32),
    )(pos8, posT, sq[:, None], sq[None, :], batch[:, None], batch[None, :])

    px = jnp.asarray(pos[:, 0])
    py = jnp.asarray(pos[:, 1])
    pz = jnp.asarray(pos[:, 2])
    src, tgt, vx, vy, vz, sqe = _sc_topk(d, px, py, pz)

    w = pl.pallas_call(
        _sqrt_body,
        out_shape=jax.ShapeDtypeStruct((n, KO), jnp.float32),
    )(sqe)

    src = src[:, :K].reshape(-1)
    tgt = tgt[:, :K].reshape(-1)
    edge_index = jnp.stack([src, tgt])
    edge_weight = w[:, :K].reshape(-1)
    edge_vec = jnp.stack([vx[:, :K], vy[:, :K], vz[:, :K]],
                         axis=-1).reshape(-1, 3)
    return edge_index, edge_weight, edge_vec


# speedup vs baseline: 1.8697x; 1.0856x over previous
import functools

import jax
import jax.numpy as jnp
from jax import lax
from jax.experimental import pallas as pl
from jax.experimental.pallas import tpu as pltpu
from jax.experimental.pallas import tpu_sc as plsc

N = 4096
K = 33
KO = 48
R2 = 25.0
ROWS = 512
RPW = 128
CAND = N + 32
INF = float("inf")
BIGI = 2**30


def _dist_body(pos8_ref, posT_ref, sq_ref, sqT_ref, b_ref, bT_ref, d_ref):
    blk = pl.program_id(0)
    dot = jnp.dot(pos8_ref[...], posT_ref[...],
                  preferred_element_type=jnp.float32)
    d2 = sq_ref[...] + sqT_ref[...] - 2.0 * dot
    d2 = jnp.maximum(d2, 0.0)
    row_ids = blk * ROWS + jax.lax.broadcasted_iota(jnp.int32, (ROWS, N), 0)
    col_ids = jax.lax.broadcasted_iota(jnp.int32, (ROWS, N), 1)
    valid = (b_ref[...] == bT_ref[...]) & (row_ids != col_ids) & (d2 <= R2)
    d_ref[...] = jnp.where(valid, d2, INF)


def _sqrt_body(x_ref, o_ref):
    o_ref[...] = jnp.sqrt(x_ref[...])


def _sc_topk_body(d_hbm, px_hbm, py_hbm, pz_hbm,
                  src_hbm, tgt_hbm, vx_hbm, vy_hbm, vz_hbm, sqe_hbm,
                  rb0, rb1, cand_i, px_v, py_v, pz_v,
                  osrc, otgt, ovx, ovy, ovz, osqe,
                  sem0, sem1):
    nc = 2
    wid = lax.axis_index("s") * nc + lax.axis_index("c")
    base = wid * RPW
    iota16 = lax.iota(jnp.int32, 16)

    pltpu.sync_copy(px_hbm, px_v)
    pltpu.sync_copy(py_hbm, py_v)
    pltpu.sync_copy(pz_hbm, pz_v)

    inf16 = jnp.full((16,), INF, jnp.float32)
    rb0[pl.ds(N, 16)] = inf16
    rb0[pl.ds(N + 16, 16)] = inf16
    rb1[pl.ds(N, 16)] = inf16
    rb1[pl.ds(N + 16, 16)] = inf16
    neg16 = jnp.full((16,), -1, jnp.int32)
    lane0 = iota16 == 0

    def process(rb, r, ri):
        @plsc.parallel_loop(0, N // 16, carry=jnp.int32(0), unroll=8)
        def cnt(c, cnt):
            v = rb[pl.ds(c * 16, 16)]
            m = v <= R2
            cs = plsc.cumsum(m.astype(jnp.int32))
            pos = cnt + cs - 1
            plsc.store_scatter(cand_i, [pos], c * 16 + iota16, mask=m)
            return cnt + cs[15]
        padpos = cnt + iota16
        plsc.store_scatter(cand_i, [padpos], N + iota16)
        plsc.store_scatter(cand_i, [padpos + 16], N + 16 + iota16)
        nv = (cnt + 15) // 16

        risplat = jnp.full((16,), ri, jnp.int32)
        osrc[ri, pl.ds(0, 16)] = neg16
        osrc[ri, pl.ds(16, 16)] = neg16
        osrc[ri, pl.ds(32, 16)] = neg16

        nk = jnp.minimum(cnt, K)

        @pl.when(cnt <= 32)
        def _():
            vi0 = cand_i[pl.ds(0, 16)]
            vi1 = cand_i[pl.ds(16, 16)]

            def ext_fast(k, _):
                vd0 = plsc.load_gather(rb, [vi0])
                vd1 = plsc.load_gather(rb, [vi1])
                ml0 = jnp.min(vd0)
                ml1 = jnp.min(vd1)
                bl0 = jnp.min(jnp.where(vd0 == ml0, vi0, BIGI))
                bl1 = jnp.min(jnp.where(vd1 == ml1, vi1, BIGI))
                lt = ml1 < ml0
                eq = ml1 == ml0
                bi = jnp.where(lt, bl1,
                               jnp.where(eq, jnp.minimum(bl0, bl1), bl0))
                bsplat = jnp.full((16,), bi, jnp.int32)
                plsc.store_scatter(rb, [bsplat], inf16, mask=lane0)
                plsc.store_scatter(osrc,
                                   [risplat, jnp.full((16,), k, jnp.int32)],
                                   bsplat, mask=lane0)
                return 0
            lax.fori_loop(0, nk, ext_fast, 0)

        @pl.when((cnt > 32) & (cnt <= 48))
        def _():
            vi0 = cand_i[pl.ds(0, 16)]
            vi1 = cand_i[pl.ds(16, 16)]
            vi2 = cand_i[pl.ds(32, 16)]

            def ext_mid(k, _):
                vd0 = plsc.load_gather(rb, [vi0])
                vd1 = plsc.load_gather(rb, [vi1])
                vd2 = plsc.load_gather(rb, [vi2])
                ml0 = jnp.min(vd0)
                ml1 = jnp.min(vd1)
                ml2 = jnp.min(vd2)
                bl0 = jnp.min(jnp.where(vd0 == ml0, vi0, BIGI))
                bl1 = jnp.min(jnp.where(vd1 == ml1, vi1, BIGI))
                bl2 = jnp.min(jnp.where(vd2 == ml2, vi2, BIGI))
                lt = ml1 < ml0
                eq = ml1 == ml0
                ma = jnp.where(lt, ml1, ml0)
                ba = jnp.where(lt, bl1,
                               jnp.where(eq, jnp.minimum(bl0, bl1), bl0))
                lt2 = ml2 < ma
                eq2 = ml2 == ma
                bi = jnp.where(lt2, bl2,
                               jnp.where(eq2, jnp.minimum(ba, bl2), ba))
                bsplat = jnp.full((16,), bi, jnp.int32)
                plsc.store_scatter(rb, [bsplat], inf16, mask=lane0)
                plsc.store_scatter(osrc,
                                   [risplat, jnp.full((16,), k, jnp.int32)],
                                   bsplat, mask=lane0)
                return 0
            lax.fori_loop(0, nk, ext_mid, 0)

        @pl.when(cnt > 48)
        def _():
            def ext_body(k, _):
                def min_body(t, carry):
                    m, bi = carry
                    vi = cand_i[pl.ds(t * 16, 16)]
                    vd = plsc.load_gather(rb, [vi])
                    ml = jnp.min(vd)
                    bl = jnp.min(jnp.where(vd == ml, vi, BIGI))
                    lt = ml < m
                    eq = ml == m
                    return (jnp.where(lt, ml, m),
                            jnp.where(lt, bl,
                                      jnp.where(eq, jnp.minimum(bi, bl), bi)))
                m, bi = lax.fori_loop(0, nv, min_body,
                                      (jnp.float32(INF), jnp.int32(BIGI)))
                bsplat = jnp.full((16,), bi, jnp.int32)
                plsc.store_scatter(rb, [bsplat], inf16, mask=lane0)
                plsc.store_scatter(osrc,
                                   [risplat, jnp.full((16,), k, jnp.int32)],
                                   bsplat, mask=lane0)
                return 0
            lax.fori_loop(0, nk, ext_body, 0)

        rsplat = jnp.full((16,), r, jnp.int32)
        rx = plsc.load_gather(px_v, [rsplat])
        ry = plsc.load_gather(py_v, [rsplat])
        rz = plsc.load_gather(pz_v, [rsplat])
        for t in range(KO // 16):
            sidx = osrc[ri, pl.ds(t * 16, 16)]
            val = sidx >= 0
            ci = jnp.maximum(sidx, 0)
            gx = plsc.load_gather(px_v, [ci])
            gy = plsc.load_gather(py_v, [ci])
            gz = plsc.load_gather(pz_v, [ci])
            zeros = jnp.zeros((16,), jnp.float32)
            vx = jnp.where(val, gx - rx, zeros)
            vy = jnp.where(val, gy - ry, zeros)
            vz = jnp.where(val, gz - rz, zeros)
            sq = vx * vx + vy * vy + vz * vz
            ovx[ri, pl.ds(t * 16, 16)] = vx
            ovy[ri, pl.ds(t * 16, 16)] = vy
            ovz[ri, pl.ds(t * 16, 16)] = vz
            osqe[ri, pl.ds(t * 16, 16)] = sq
            otgt[ri, pl.ds(t * 16, 16)] = jnp.where(val, r, -1)

    pltpu.async_copy(d_hbm.at[base], rb0.at[pl.ds(0, N)], sem0)
    def row_step(i, _):
        r0 = base + 2 * i
        pltpu.make_async_copy(d_hbm.at[r0], rb0.at[pl.ds(0, N)], sem0).wait()
        r1 = r0 + 1
        pltpu.async_copy(d_hbm.at[r1], rb1.at[pl.ds(0, N)], sem1)
        process(rb0, r0, 2 * i)
        pltpu.make_async_copy(d_hbm.at[r1], rb1.at[pl.ds(0, N)], sem1).wait()
        r2 = jnp.minimum(r0 + 2, N - 1)
        pltpu.async_copy(d_hbm.at[r2], rb0.at[pl.ds(0, N)], sem0)
        process(rb1, r1, 2 * i + 1)
        return 0
    lax.fori_loop(0, RPW // 2, row_step, 0)
    pltpu.make_async_copy(d_hbm.at[0], rb0.at[pl.ds(0, N)], sem0).wait()

    pltpu.sync_copy(osrc, src_hbm.at[pl.ds(base, RPW)])
    pltpu.sync_copy(otgt, tgt_hbm.at[pl.ds(base, RPW)])
    pltpu.sync_copy(ovx, vx_hbm.at[pl.ds(base, RPW)])
    pltpu.sync_copy(ovy, vy_hbm.at[pl.ds(base, RPW)])
    pltpu.sync_copy(ovz, vz_hbm.at[pl.ds(base, RPW)])
    pltpu.sync_copy(osqe, sqe_hbm.at[pl.ds(base, RPW)])


def _sc_topk(d, px, py, pz):
    mesh = plsc.VectorSubcoreMesh(core_axis_name="c", subcore_axis_name="s")
    f32 = jnp.float32
    i32 = jnp.int32
    fn = functools.partial(
        pl.kernel,
        mesh=mesh,
        compiler_params=pltpu.CompilerParams(needs_layout_passes=False),
        out_type=[
            jax.ShapeDtypeStruct((N, KO), i32),
            jax.ShapeDtypeStruct((N, KO), i32),
            jax.ShapeDtypeStruct((N, KO), f32),
            jax.ShapeDtypeStruct((N, KO), f32),
            jax.ShapeDtypeStruct((N, KO), f32),
            jax.ShapeDtypeStruct((N, KO), f32),
        ],
        scratch_types=[
            pltpu.VMEM((CAND,), f32),
            pltpu.VMEM((CAND,), f32),
            pltpu.VMEM((CAND,), i32),
            pltpu.VMEM((N,), f32),
            pltpu.VMEM((N,), f32),
            pltpu.VMEM((N,), f32),
            pltpu.VMEM((RPW, KO), i32),
            pltpu.VMEM((RPW, KO), i32),
            pltpu.VMEM((RPW, KO), f32),
            pltpu.VMEM((RPW, KO), f32),
            pltpu.VMEM((RPW, KO), f32),
            pltpu.VMEM((RPW, KO), f32),
            pltpu.SemaphoreType.DMA,
            pltpu.SemaphoreType.DMA,
        ],
    )(_sc_topk_body)
    return fn(d, px, py, pz)


@jax.jit
def kernel(pos, batch):
    n = pos.shape[0]
    sq = jnp.sum(pos * pos, axis=1)
    pos8 = jnp.pad(pos, ((0, 0), (0, 5)))
    posT = pos8.T
    d = pl.pallas_call(
        _dist_body,
        grid=(n // ROWS,),
        in_specs=[
            pl.BlockSpec((ROWS, 8), lambda i: (i, 0)),
            pl.BlockSpec((8, N), lambda i: (0, 0)),
            pl.BlockSpec((ROWS, 1), lambda i: (i, 0)),
            pl.BlockSpec((1, N), lambda i: (0, 0)),
            pl.BlockSpec((ROWS, 1), lambda i: (i, 0)),
            pl.BlockSpec((1, N), lambda i: (0, 0)),
        ],
        out_specs=pl.BlockSpec((ROWS, N), lambda i: (i, 0)),
        out_shape=jax.ShapeDtypeStruct((n, n), jnp.float32),
    )(pos8, posT, sq[:, None], sq[None, :], batch[:, None], batch[None, :])

    px = jnp.asarray(pos[:, 0])
    py = jnp.asarray(pos[:, 1])
    pz = jnp.asarray(pos[:, 2])
    src, tgt, vx, vy, vz, sqe = _sc_topk(d, px, py, pz)

    w = pl.pallas_call(
        _sqrt_body,
        out_shape=jax.ShapeDtypeStruct((n, KO), jnp.float32),
    )(sqe)

    src = src[:, :K].reshape(-1)
    tgt = tgt[:, :K].reshape(-1)
    edge_index = jnp.stack([src, tgt])
    edge_weight = w[:, :K].reshape(-1)
    edge_vec = jnp.stack([vx[:, :K], vy[:, :K], vz[:, :K]],
                         axis=-1).reshape(-1, 3)
    return edge_index, edge_weight, edge_vec
